# Initial kernel scaffold; baseline (speedup 1.0000x reference)
import jax
import jax.numpy as jnp
from jax.experimental import pallas as pl


def kernel(s, v, p, edge_index_local, d_local, a_local, r_norm_local, e_local, edge_index_global, d_global, a_global, r_norm_global, e_global, batch, W_msg, b_msg, W_upd, gamma_s, beta_s):
    raise NotImplementedError("write your pallas kernel here")


# speedup vs baseline: 9.9370x; 9.9370x over previous
import functools

import jax
import jax.numpy as jnp
from jax.experimental import pallas as pl
from jax.experimental.pallas import tpu as pltpu

_PREC = jax.lax.Precision.HIGHEST


def _dot(a, b, prec=_PREC):
    return jax.lax.dot_general(a, b, (((1,), (0,)), ((), ())),
                               preferred_element_type=jnp.float32,
                               precision=prec)


def _pre_kernel(s_ref, v_ref, p_ref, g_ref, b_ref, tbl_ref):
    n = s_ref.shape[0]
    s = s_ref[...]
    mu = jnp.mean(s, axis=1, keepdims=True)
    var = jnp.mean((s - mu) * (s - mu), axis=1, keepdims=True)
    sn = (s - mu) / jnp.sqrt(var + 1e-5) * g_ref[...] + b_ref[...]
    v2 = v_ref[...]
    nv = v2.shape[1]
    rms = jnp.sqrt(jnp.sum(v2 * v2, axis=1, keepdims=True) / (nv // 3) + 1e-6)
    vn = v2 / rms
    pad = jnp.zeros((n, 128 - 64 - nv - 3), dtype=jnp.float32)
    tbl_ref[...] = jnp.concatenate([sn, vn, p_ref[:, 0:3], pad], axis=1)


def _edge_kernel(src_c_ref, dst_c_ref, dst_r_ref, e_ref, geom_ref, tbl_ref,
                 w1_ref, w2_ref, w3_ref, w45b_ref, wupd_ref, gb_ref,
                 batch_ref, eout_ref, tbln_ref, svp_ref, acc_ref,
                 *, nblk, sdim, vdim, edim, nb, layer0):
    i = pl.program_id(0)
    n = tbl_ref.shape[0]
    t = e_ref.shape[0]
    nv = 3 * vdim

    tbl = tbl_ref[...]
    src_c = src_c_ref[0]
    dst_c = dst_c_ref[0]
    dst_r = dst_r_ref[0]

    iota_tn = jax.lax.broadcasted_iota(jnp.int32, (t, n), 1)
    ohs_t = (iota_tn == src_c).astype(jnp.float32)
    ohd_t = (iota_tn == dst_c).astype(jnp.float32)
    iota_nt = jax.lax.broadcasted_iota(jnp.int32, (n, t), 0)
    ohd = (iota_nt == dst_r).astype(jnp.float32)

    gs = _dot(ohs_t, tbl)
    gd = _dot(ohd_t, tbl)

    if layer0:
        g8 = geom_ref[...]
        rn = g8[:, 0:3]
        d = g8[:, 3:4]
        a = g8[:, 4:5]
    else:
        ps = gs[:, 64 + nv:64 + nv + 3]
        pd = gd[:, 64 + nv:64 + nv + 3]
        rv = pd - ps
        d = jnp.sqrt(jnp.clip(jnp.sum(rv * rv, axis=1, keepdims=True),
                              1e-6, None))
        a = jnp.sum(ps * pd, axis=1, keepdims=True)
        rn = rv / d

    eb = e_ref[...]
    m = (_dot(gs[:, 0:sdim], w1_ref[...]) + _dot(gd[:, 0:sdim], w2_ref[...])
         + _dot(eb, w3_ref[...])
         + d * w45b_ref[0:1, :] + a * w45b_ref[1:2, :] + w45b_ref[2:3, :])

    ms = jax.nn.silu(m[:, 0:sdim])
    gv = m[:, sdim:sdim + vdim]
    gr = m[:, sdim + vdim:sdim + 2 * vdim]
    enew = m[:, sdim + 2 * vdim:sdim + 2 * vdim + edim]
    gp = m[:, sdim + 2 * vdim + edim:sdim + 2 * vdim + edim + 1]

    eout_ref[...] = jax.nn.silu(enew)

    vs = gs[:, sdim:sdim + nv]
    gv3 = jnp.concatenate([gv, gv, gv], axis=1)
    gr3 = jnp.concatenate([gr, gr, gr], axis=1)
    rexp = jnp.concatenate(
        [jnp.broadcast_to(rn[:, c:c + 1], (t, vdim)) for c in range(3)],
        axis=1)
    mv = vs * gv3 + rexp * gr3
    pt = rn * jnp.tanh(gp)

    ones = jnp.ones((t, 1), dtype=jnp.float32)
    zpad = jnp.zeros((t, 128 - sdim - nv - 3 - 1), dtype=jnp.float32)
    payload = jnp.concatenate([ms, mv, pt, ones, zpad], axis=1)

    contrib = jax.lax.dot_general(ohd, payload, (((1,), (0,)), ((), ())),
                                  preferred_element_type=jnp.float32,
                                  precision=_PREC)

    @pl.when(i == 0)
    def _():
        acc_ref[...] = jnp.zeros_like(acc_ref)

    acc_ref[...] += contrib

    @pl.when(i == nblk - 1)
    def _():
        acc = acc_ref[...]
        ccol = sdim + nv + 3
        cnt = acc[:, ccol:ccol + 1]
        inv = 1.0 / jnp.maximum(cnt, 1.0)
        sn = tbl[:, 0:sdim]
        vn = tbl[:, sdim:sdim + nv]
        pcur = tbl[:, sdim + nv:sdim + nv + 3]
        s_next = sn + _dot(acc[:, 0:sdim] * inv, wupd_ref[...])
        v_next = vn + acc[:, sdim:sdim + nv] * inv
        p_mid = pcur + acc[:, sdim + nv:sdim + nv + 3] * inv

        iota_b = jax.lax.broadcasted_iota(jnp.int32, (n, 128), 1)
        ohb = (iota_b == batch_ref[...]).astype(jnp.float32)
        p4 = jnp.concatenate([p_mid, jnp.ones((n, 1), jnp.float32)], axis=1)
        bs = jax.lax.dot_general(ohb, p4, (((0,), (0,)), ((), ())),
                                 preferred_element_type=jnp.float32,
                                 precision=_PREC)
        minv = 1.0 / jnp.maximum(bs[:, 3:4], 1.0)
        p_next = p_mid - _dot(ohb, bs[:, 0:3] * minv)

        zc = jnp.zeros((n, 128 - sdim - nv - 3), dtype=jnp.float32)
        svp_ref[...] = jnp.concatenate([s_next, v_next, p_next, zc], axis=1)

        mu = jnp.mean(s_next, axis=1, keepdims=True)
        var = jnp.mean((s_next - mu) * (s_next - mu), axis=1, keepdims=True)
        sn2 = ((s_next - mu) / jnp.sqrt(var + 1e-5) * gb_ref[0:1, 0:sdim]
               + gb_ref[1:2, 0:sdim])
        rms = jnp.sqrt(jnp.sum(v_next * v_next, axis=1, keepdims=True)
                       / vdim + 1e-6)
        vn2 = v_next / rms
        tbln_ref[...] = jnp.concatenate([sn2, vn2, p_next, zc], axis=1)


def _pad_cols(x, w=128):
    return jnp.pad(x, ((0, 0), (0, w - x.shape[1])))


def kernel(s, v, p, edge_index_local, d_local, a_local, r_norm_local,
           e_local, edge_index_global, d_global, a_global, r_norm_global,
           e_global, batch, W_msg, b_msg, W_upd, gamma_s, beta_s):
    n, sdim = s.shape
    vdim = v.shape[2]
    nv = 3 * vdim
    eg = e_global.shape[0]
    edim = e_global.shape[1]
    nl = W_msg.shape[0]
    nb = 16

    t = min(1024, eg)
    nblk = eg // t

    f32 = jnp.float32
    v2 = v.reshape(n, nv).astype(f32)
    p8 = jnp.pad(p.astype(f32), ((0, 0), (0, 5)))
    src = edge_index_global[0].astype(jnp.int32)
    dst = edge_index_global[1].astype(jnp.int32)
    src_c = src.reshape(nblk, t, 1)
    dst_c = dst.reshape(nblk, t, 1)
    dst_r = dst.reshape(nblk, 1, t)
    geom0 = jnp.concatenate(
        [r_norm_global.astype(f32), d_global[:, None].astype(f32),
         a_global[:, None].astype(f32), jnp.zeros((eg, 3), f32)],
        axis=1)

    tbl = pl.pallas_call(
        _pre_kernel,
        out_shape=jax.ShapeDtypeStruct((n, 128), f32),
    )(s.astype(f32), v2, p8, gamma_s[0][None, :].astype(f32),
      beta_s[0][None, :].astype(f32))

    batch2 = batch.reshape(n, 1).astype(jnp.int32)

    e_cur = e_global.astype(f32)
    svp = None
    for i in range(nl):
        W = W_msg[i].astype(f32)
        w1p = _pad_cols(W[0:sdim])
        w2p = _pad_cols(W[sdim:2 * sdim])
        w3p = _pad_cols(W[2 * sdim:2 * sdim + edim])
        w45b = jnp.pad(
            jnp.stack([W[2 * sdim + edim], W[2 * sdim + edim + 1],
                       b_msg[i].astype(f32)]),
            ((0, 5), (0, 128 - W.shape[1])))
        gbn = jnp.pad(
            jnp.stack([gamma_s[(i + 1) % nl], beta_s[(i + 1) % nl]]),
            ((0, 6), (0, 128 - sdim))).astype(f32)

        body = functools.partial(_edge_kernel, nblk=nblk, sdim=sdim,
                                 vdim=vdim, edim=edim, nb=nb,
                                 layer0=(i == 0))
        e_cur, tbl, svp = pl.pallas_call(
            body,
            grid=(nblk,),
            in_specs=[
                pl.BlockSpec((1, t, 1), lambda i: (i, 0, 0)),
                pl.BlockSpec((1, t, 1), lambda i: (i, 0, 0)),
                pl.BlockSpec((1, 1, t), lambda i: (i, 0, 0)),
                pl.BlockSpec((t, edim), lambda i: (i, 0)),
                pl.BlockSpec((t, 8), lambda i: (i, 0)),
                pl.BlockSpec((n, 128), lambda i: (0, 0)),
                pl.BlockSpec((sdim, 128), lambda i: (0, 0)),
                pl.BlockSpec((sdim, 128), lambda i: (0, 0)),
                pl.BlockSpec((edim, 128), lambda i: (0, 0)),
                pl.BlockSpec((8, 128), lambda i: (0, 0)),
                pl.BlockSpec((sdim, sdim), lambda i: (0, 0)),
                pl.BlockSpec((8, 128), lambda i: (0, 0)),
                pl.BlockSpec((n, 1), lambda i: (0, 0)),
            ],
            out_specs=[
                pl.BlockSpec((t, edim), lambda i: (i, 0)),
                pl.BlockSpec((n, 128), lambda i: (0, 0)),
                pl.BlockSpec((n, 128), lambda i: (0, 0)),
            ],
            out_shape=[
                jax.ShapeDtypeStruct((eg, edim), f32),
                jax.ShapeDtypeStruct((n, 128), f32),
                jax.ShapeDtypeStruct((n, 128), f32),
            ],
            scratch_shapes=[pltpu.VMEM((n, 128), f32)],
        )(src_c, dst_c, dst_r, e_cur, geom0, tbl, w1p, w2p, w3p, w45b,
          W_upd[i].astype(f32), gbn, batch2)

    s_o = svp[:, 0:sdim]
    v_o = svp[:, sdim:sdim + nv].reshape(n, 3, vdim)
    p_o = svp[:, sdim + nv:sdim + nv + 3]
    return s_o, v_o, e_cur, p_o


# speedup vs baseline: 12.7620x; 1.2843x over previous
import functools

import jax
import jax.numpy as jnp
from jax.experimental import pallas as pl
from jax.experimental.pallas import tpu as pltpu

_PREC = jax.lax.Precision.HIGHEST
_HIGH = jax.lax.Precision.HIGHEST
_FAST = jax.lax.Precision.DEFAULT


def _dot(a, b, prec=_PREC):
    return jax.lax.dot_general(a, b, (((1,), (0,)), ((), ())),
                               preferred_element_type=jnp.float32,
                               precision=prec)


def _pre_kernel(s_ref, v_ref, p_ref, g_ref, b_ref, tbl_ref):
    n = s_ref.shape[0]
    s = s_ref[...]
    mu = jnp.mean(s, axis=1, keepdims=True)
    var = jnp.mean((s - mu) * (s - mu), axis=1, keepdims=True)
    sn = (s - mu) / jnp.sqrt(var + 1e-5) * g_ref[...] + b_ref[...]
    v2 = v_ref[...]
    nv = v2.shape[1]
    rms = jnp.sqrt(jnp.sum(v2 * v2, axis=1, keepdims=True) / (nv // 3) + 1e-6)
    vn = v2 / rms
    pad = jnp.zeros((n, 128 - 64 - nv - 3), dtype=jnp.float32)
    tbl_ref[...] = jnp.concatenate([sn, vn, p_ref[:, 0:3], pad], axis=1)


def _edge_kernel(src_c_ref, dst_c_ref, dst_r_ref, e_ref, geom_ref, tbl_ref,
                 w1_ref, w2_ref, w3_ref, w45b_ref, wupd_ref, gb_ref,
                 batch_ref, eout_ref, tbln_ref, svp_ref, acc_ref,
                 *, nblk, sdim, vdim, edim, nb, layer0):
    i = pl.program_id(0)
    n = tbl_ref.shape[0]
    t = e_ref.shape[0]
    nv = 3 * vdim

    tbl = tbl_ref[...]
    src_c = src_c_ref[0]
    dst_c = dst_c_ref[0]
    dst_r = dst_r_ref[0]

    iota_tn = jax.lax.broadcasted_iota(jnp.int32, (t, n), 1)
    ohs_t = (iota_tn == src_c).astype(jnp.bfloat16)
    ohd_t = (iota_tn == dst_c).astype(jnp.bfloat16)
    iota_nt = jax.lax.broadcasted_iota(jnp.int32, (n, t), 0)
    ohd = (iota_nt == dst_r).astype(jnp.bfloat16)

    tblb = tbl.astype(jnp.bfloat16)
    gs = _dot(ohs_t, tblb, _FAST)
    gd = _dot(ohd_t, tblb, _FAST)

    if layer0:
        g8 = geom_ref[...]
        rn = g8[:, 0:3]
        d = g8[:, 3:4]
        a = g8[:, 4:5]
    else:
        ptab = tbl[:, sdim + nv:sdim + nv + 8]
        ps8 = _dot(ohs_t.astype(jnp.float32), ptab, _HIGH)
        pd8 = _dot(ohd_t.astype(jnp.float32), ptab, _HIGH)
        ps = ps8[:, 0:3]
        pd = pd8[:, 0:3]
        rv = pd - ps
        d = jnp.sqrt(jnp.clip(jnp.sum(rv * rv, axis=1, keepdims=True),
                              1e-6, None))
        a = jnp.sum(ps * pd, axis=1, keepdims=True)
        rn = rv / d

    eb = e_ref[...]
    m = (_dot(gs[:, 0:sdim], w1_ref[...], _HIGH)
         + _dot(gd[:, 0:sdim], w2_ref[...], _HIGH)
         + _dot(eb, w3_ref[...], _HIGH)
         + d * w45b_ref[0:1, :] + a * w45b_ref[1:2, :] + w45b_ref[2:3, :])

    ms = jax.nn.silu(m[:, 0:sdim])
    gv = m[:, sdim:sdim + vdim]
    gr = m[:, sdim + vdim:sdim + 2 * vdim]
    enew = m[:, sdim + 2 * vdim:sdim + 2 * vdim + edim]
    gp = m[:, sdim + 2 * vdim + edim:sdim + 2 * vdim + edim + 1]

    eout_ref[...] = jax.nn.silu(enew)

    vs = gs[:, sdim:sdim + nv]
    gv3 = jnp.concatenate([gv, gv, gv], axis=1)
    gr3 = jnp.concatenate([gr, gr, gr], axis=1)
    rexp = jnp.concatenate(
        [jnp.broadcast_to(rn[:, c:c + 1], (t, vdim)) for c in range(3)],
        axis=1)
    mv = vs * gv3 + rexp * gr3
    pt = rn * jnp.tanh(gp)

    ones = jnp.ones((t, 1), dtype=jnp.float32)
    zpad = jnp.zeros((t, 128 - sdim - nv - 3 - 1), dtype=jnp.float32)
    payload = jnp.concatenate([ms, mv, pt, ones, zpad], axis=1)

    contrib = jax.lax.dot_general(ohd, payload.astype(jnp.bfloat16),
                                  (((1,), (0,)), ((), ())),
                                  preferred_element_type=jnp.float32,
                                  precision=_FAST)

    @pl.when(i == 0)
    def _():
        acc_ref[...] = jnp.zeros_like(acc_ref)

    acc_ref[...] += contrib

    @pl.when(i == nblk - 1)
    def _():
        acc = acc_ref[...]
        ccol = sdim + nv + 3
        cnt = acc[:, ccol:ccol + 1]
        inv = 1.0 / jnp.maximum(cnt, 1.0)
        sn = tbl[:, 0:sdim]
        vn = tbl[:, sdim:sdim + nv]
        pcur = tbl[:, sdim + nv:sdim + nv + 3]
        s_next = sn + _dot(acc[:, 0:sdim] * inv, wupd_ref[...])
        v_next = vn + acc[:, sdim:sdim + nv] * inv
        p_mid = pcur + acc[:, sdim + nv:sdim + nv + 3] * inv

        iota_b = jax.lax.broadcasted_iota(jnp.int32, (n, 128), 1)
        ohb = (iota_b == batch_ref[...]).astype(jnp.float32)
        p4 = jnp.concatenate([p_mid, jnp.ones((n, 1), jnp.float32)], axis=1)
        bs = jax.lax.dot_general(ohb, p4, (((0,), (0,)), ((), ())),
                                 preferred_element_type=jnp.float32,
                                 precision=_PREC)
        minv = 1.0 / jnp.maximum(bs[:, 3:4], 1.0)
        p_next = p_mid - _dot(ohb, bs[:, 0:3] * minv)

        zc = jnp.zeros((n, 128 - sdim - nv - 3), dtype=jnp.float32)
        svp_ref[...] = jnp.concatenate([s_next, v_next, p_next, zc], axis=1)

        mu = jnp.mean(s_next, axis=1, keepdims=True)
        var = jnp.mean((s_next - mu) * (s_next - mu), axis=1, keepdims=True)
        sn2 = ((s_next - mu) / jnp.sqrt(var + 1e-5) * gb_ref[0:1, 0:sdim]
               + gb_ref[1:2, 0:sdim])
        rms = jnp.sqrt(jnp.sum(v_next * v_next, axis=1, keepdims=True)
                       / vdim + 1e-6)
        vn2 = v_next / rms
        tbln_ref[...] = jnp.concatenate([sn2, vn2, p_next, zc], axis=1)


def _pad_cols(x, w=128):
    return jnp.pad(x, ((0, 0), (0, w - x.shape[1])))


def kernel(s, v, p, edge_index_local, d_local, a_local, r_norm_local,
           e_local, edge_index_global, d_global, a_global, r_norm_global,
           e_global, batch, W_msg, b_msg, W_upd, gamma_s, beta_s):
    n, sdim = s.shape
    vdim = v.shape[2]
    nv = 3 * vdim
    eg = e_global.shape[0]
    edim = e_global.shape[1]
    nl = W_msg.shape[0]
    nb = 16

    t = min(1024, eg)
    nblk = eg // t

    f32 = jnp.float32
    v2 = v.reshape(n, nv).astype(f32)
    p8 = jnp.pad(p.astype(f32), ((0, 0), (0, 5)))
    src = edge_index_global[0].astype(jnp.int32)
    dst = edge_index_global[1].astype(jnp.int32)
    src_c = src.reshape(nblk, t, 1)
    dst_c = dst.reshape(nblk, t, 1)
    dst_r = dst.reshape(nblk, 1, t)
    geom0 = jnp.concatenate(
        [r_norm_global.astype(f32), d_global[:, None].astype(f32),
         a_global[:, None].astype(f32), jnp.zeros((eg, 3), f32)],
        axis=1)

    tbl = pl.pallas_call(
        _pre_kernel,
        out_shape=jax.ShapeDtypeStruct((n, 128), f32),
    )(s.astype(f32), v2, p8, gamma_s[0][None, :].astype(f32),
      beta_s[0][None, :].astype(f32))

    batch2 = batch.reshape(n, 1).astype(jnp.int32)

    e_cur = e_global.astype(f32)
    svp = None
    for i in range(nl):
        W = W_msg[i].astype(f32)
        w1p = _pad_cols(W[0:sdim])
        w2p = _pad_cols(W[sdim:2 * sdim])
        w3p = _pad_cols(W[2 * sdim:2 * sdim + edim])
        w45b = jnp.pad(
            jnp.stack([W[2 * sdim + edim], W[2 * sdim + edim + 1],
                       b_msg[i].astype(f32)]),
            ((0, 5), (0, 128 - W.shape[1])))
        gbn = jnp.pad(
            jnp.stack([gamma_s[(i + 1) % nl], beta_s[(i + 1) % nl]]),
            ((0, 6), (0, 128 - sdim))).astype(f32)

        body = functools.partial(_edge_kernel, nblk=nblk, sdim=sdim,
                                 vdim=vdim, edim=edim, nb=nb,
                                 layer0=(i == 0))
        e_cur, tbl, svp = pl.pallas_call(
            body,
            grid=(nblk,),
            in_specs=[
                pl.BlockSpec((1, t, 1), lambda i: (i, 0, 0)),
                pl.BlockSpec((1, t, 1), lambda i: (i, 0, 0)),
                pl.BlockSpec((1, 1, t), lambda i: (i, 0, 0)),
                pl.BlockSpec((t, edim), lambda i: (i, 0)),
                pl.BlockSpec((t, 8), lambda i: (i, 0)),
                pl.BlockSpec((n, 128), lambda i: (0, 0)),
                pl.BlockSpec((sdim, 128), lambda i: (0, 0)),
                pl.BlockSpec((sdim, 128), lambda i: (0, 0)),
                pl.BlockSpec((edim, 128), lambda i: (0, 0)),
                pl.BlockSpec((8, 128), lambda i: (0, 0)),
                pl.BlockSpec((sdim, sdim), lambda i: (0, 0)),
                pl.BlockSpec((8, 128), lambda i: (0, 0)),
                pl.BlockSpec((n, 1), lambda i: (0, 0)),
            ],
            out_specs=[
                pl.BlockSpec((t, edim), lambda i: (i, 0)),
                pl.BlockSpec((n, 128), lambda i: (0, 0)),
                pl.BlockSpec((n, 128), lambda i: (0, 0)),
            ],
            out_shape=[
                jax.ShapeDtypeStruct((eg, edim), f32),
                jax.ShapeDtypeStruct((n, 128), f32),
                jax.ShapeDtypeStruct((n, 128), f32),
            ],
            scratch_shapes=[pltpu.VMEM((n, 128), f32)],
        )(src_c, dst_c, dst_r, e_cur, geom0, tbl, w1p, w2p, w3p, w45b,
          W_upd[i].astype(f32), gbn, batch2)

    s_o = svp[:, 0:sdim]
    v_o = svp[:, sdim:sdim + nv].reshape(n, 3, vdim)
    p_o = svp[:, sdim + nv:sdim + nv + 3]
    return s_o, v_o, e_cur, p_o


# speedup vs baseline: 14.6346x; 1.1467x over previous
import functools

import jax
import jax.numpy as jnp
from jax.experimental import pallas as pl
from jax.experimental.pallas import tpu as pltpu

_PREC = jax.lax.Precision.HIGHEST
_HIGH = jax.lax.Precision.HIGHEST
_FAST = jax.lax.Precision.DEFAULT


def _dot(a, b, prec=_PREC):
    return jax.lax.dot_general(a, b, (((1,), (0,)), ((), ())),
                               preferred_element_type=jnp.float32,
                               precision=prec)


def _pre_kernel(s_ref, v_ref, p_ref, g_ref, b_ref, tbl_ref):
    n = s_ref.shape[0]
    s = s_ref[...]
    mu = jnp.mean(s, axis=1, keepdims=True)
    var = jnp.mean((s - mu) * (s - mu), axis=1, keepdims=True)
    sn = (s - mu) / jnp.sqrt(var + 1e-5) * g_ref[...] + b_ref[...]
    v2 = v_ref[...]
    nv = v2.shape[1]
    rms = jnp.sqrt(jnp.sum(v2 * v2, axis=1, keepdims=True) / (nv // 3) + 1e-6)
    vn = v2 / rms
    p3 = p_ref[:, 0:3]
    phi = p3.astype(jnp.bfloat16).astype(jnp.float32)
    plo = p3 - phi
    pad = jnp.zeros((n, 128 - 64 - nv - 6), dtype=jnp.float32)
    tbl_ref[...] = jnp.concatenate([sn, vn, phi, plo, pad], axis=1)


def _edge_kernel(src_c_ref, dst_c_ref, dst_r_ref, e_ref, geom_ref, tbl_ref,
                 w1_ref, w2_ref, w3_ref, w45b_ref, wupd_ref, gb_ref,
                 batch_ref, eout_ref, tbln_ref, svp_ref, acc_ref,
                 *, nblk, sdim, vdim, edim, nb, layer0):
    i = pl.program_id(0)
    n = tbl_ref.shape[0]
    t = e_ref.shape[0]
    nv = 3 * vdim

    tbl = tbl_ref[...]
    src_c = src_c_ref[0]
    dst_c = dst_c_ref[0]
    dst_r = dst_r_ref[0]

    iota_tn = jax.lax.broadcasted_iota(jnp.int32, (t, n), 1)
    ohs_t = (iota_tn == src_c).astype(jnp.bfloat16)
    ohd_t = (iota_tn == dst_c).astype(jnp.bfloat16)
    iota_nt = jax.lax.broadcasted_iota(jnp.int32, (n, t), 0)
    ohd = (iota_nt == dst_r).astype(jnp.bfloat16)

    tblb = tbl.astype(jnp.bfloat16)
    gs = _dot(ohs_t, tblb, _FAST)
    gd = _dot(ohd_t, tblb, _FAST)

    if layer0:
        g8 = geom_ref[...]
        rn = g8[:, 0:3]
        d = g8[:, 3:4]
        a = g8[:, 4:5]
    else:
        ps = gs[:, sdim + nv:sdim + nv + 3] + gs[:, sdim + nv + 3:sdim + nv + 6]
        pd = gd[:, sdim + nv:sdim + nv + 3] + gd[:, sdim + nv + 3:sdim + nv + 6]
        rv = pd - ps
        d = jnp.sqrt(jnp.clip(jnp.sum(rv * rv, axis=1, keepdims=True),
                              1e-6, None))
        a = jnp.sum(ps * pd, axis=1, keepdims=True)
        rn = rv / d

    eb = e_ref[...]
    m = (_dot(gs[:, 0:sdim], w1_ref[...], _HIGH)
         + _dot(gd[:, 0:sdim], w2_ref[...], _HIGH)
         + _dot(eb, w3_ref[...], _HIGH)
         + d * w45b_ref[0:1, :] + a * w45b_ref[1:2, :] + w45b_ref[2:3, :])

    ms = jax.nn.silu(m[:, 0:sdim])
    gv = m[:, sdim:sdim + vdim]
    gr = m[:, sdim + vdim:sdim + 2 * vdim]
    enew = m[:, sdim + 2 * vdim:sdim + 2 * vdim + edim]
    gp = m[:, sdim + 2 * vdim + edim:sdim + 2 * vdim + edim + 1]

    eout_ref[...] = jax.nn.silu(enew)

    vs = gs[:, sdim:sdim + nv]
    gv3 = jnp.concatenate([gv, gv, gv], axis=1)
    gr3 = jnp.concatenate([gr, gr, gr], axis=1)
    rexp = jnp.concatenate(
        [jnp.broadcast_to(rn[:, c:c + 1], (t, vdim)) for c in range(3)],
        axis=1)
    mv = vs * gv3 + rexp * gr3
    pt = rn * jnp.tanh(gp)

    ones = jnp.ones((t, 1), dtype=jnp.float32)
    zpad = jnp.zeros((t, 128 - sdim - nv - 3 - 1), dtype=jnp.float32)
    payload = jnp.concatenate([ms, mv, pt, ones, zpad], axis=1)

    contrib = jax.lax.dot_general(ohd, payload.astype(jnp.bfloat16),
                                  (((1,), (0,)), ((), ())),
                                  preferred_element_type=jnp.float32,
                                  precision=_FAST)

    @pl.when(i == 0)
    def _():
        acc_ref[...] = jnp.zeros_like(acc_ref)

    acc_ref[...] += contrib

    @pl.when(i == nblk - 1)
    def _():
        acc = acc_ref[...]
        ccol = sdim + nv + 3
        cnt = acc[:, ccol:ccol + 1]
        inv = 1.0 / jnp.maximum(cnt, 1.0)
        sn = tbl[:, 0:sdim]
        vn = tbl[:, sdim:sdim + nv]
        pcur = (tbl[:, sdim + nv:sdim + nv + 3]
                + tbl[:, sdim + nv + 3:sdim + nv + 6])
        s_next = sn + _dot(acc[:, 0:sdim] * inv, wupd_ref[...])
        v_next = vn + acc[:, sdim:sdim + nv] * inv
        p_mid = pcur + acc[:, sdim + nv:sdim + nv + 3] * inv

        iota_b = jax.lax.broadcasted_iota(jnp.int32, (n, 128), 1)
        ohb = (iota_b == batch_ref[...]).astype(jnp.float32)
        p4 = jnp.concatenate([p_mid, jnp.ones((n, 1), jnp.float32)], axis=1)
        bs = jax.lax.dot_general(ohb, p4, (((0,), (0,)), ((), ())),
                                 preferred_element_type=jnp.float32,
                                 precision=_PREC)
        minv = 1.0 / jnp.maximum(bs[:, 3:4], 1.0)
        p_next = p_mid - _dot(ohb, bs[:, 0:3] * minv)

        zc = jnp.zeros((n, 128 - sdim - nv - 3), dtype=jnp.float32)
        svp_ref[...] = jnp.concatenate([s_next, v_next, p_next, zc], axis=1)

        mu = jnp.mean(s_next, axis=1, keepdims=True)
        var = jnp.mean((s_next - mu) * (s_next - mu), axis=1, keepdims=True)
        sn2 = ((s_next - mu) / jnp.sqrt(var + 1e-5) * gb_ref[0:1, 0:sdim]
               + gb_ref[1:2, 0:sdim])
        rms = jnp.sqrt(jnp.sum(v_next * v_next, axis=1, keepdims=True)
                       / vdim + 1e-6)
        vn2 = v_next / rms
        phi = p_next.astype(jnp.bfloat16).astype(jnp.float32)
        plo = p_next - phi
        zc2 = jnp.zeros((n, 128 - sdim - nv - 6), dtype=jnp.float32)
        tbln_ref[...] = jnp.concatenate([sn2, vn2, phi, plo, zc2], axis=1)


def _pad_cols(x, w=128):
    return jnp.pad(x, ((0, 0), (0, w - x.shape[1])))


def kernel(s, v, p, edge_index_local, d_local, a_local, r_norm_local,
           e_local, edge_index_global, d_global, a_global, r_norm_global,
           e_global, batch, W_msg, b_msg, W_upd, gamma_s, beta_s):
    n, sdim = s.shape
    vdim = v.shape[2]
    nv = 3 * vdim
    eg = e_global.shape[0]
    edim = e_global.shape[1]
    nl = W_msg.shape[0]
    nb = 16

    t = min(1024, eg)
    nblk = eg // t

    f32 = jnp.float32
    v2 = v.reshape(n, nv).astype(f32)
    p8 = jnp.pad(p.astype(f32), ((0, 0), (0, 5)))
    src = edge_index_global[0].astype(jnp.int32)
    dst = edge_index_global[1].astype(jnp.int32)
    src_c = src.reshape(nblk, t, 1)
    dst_c = dst.reshape(nblk, t, 1)
    dst_r = dst.reshape(nblk, 1, t)
    geom0 = jnp.concatenate(
        [r_norm_global.astype(f32), d_global[:, None].astype(f32),
         a_global[:, None].astype(f32), jnp.zeros((eg, 3), f32)],
        axis=1)

    tbl = pl.pallas_call(
        _pre_kernel,
        out_shape=jax.ShapeDtypeStruct((n, 128), f32),
    )(s.astype(f32), v2, p8, gamma_s[0][None, :].astype(f32),
      beta_s[0][None, :].astype(f32))

    batch2 = batch.reshape(n, 1).astype(jnp.int32)

    e_cur = e_global.astype(f32)
    svp = None
    for i in range(nl):
        W = W_msg[i].astype(f32)
        w1p = _pad_cols(W[0:sdim])
        w2p = _pad_cols(W[sdim:2 * sdim])
        w3p = _pad_cols(W[2 * sdim:2 * sdim + edim])
        w45b = jnp.pad(
            jnp.stack([W[2 * sdim + edim], W[2 * sdim + edim + 1],
                       b_msg[i].astype(f32)]),
            ((0, 5), (0, 128 - W.shape[1])))
        gbn = jnp.pad(
            jnp.stack([gamma_s[(i + 1) % nl], beta_s[(i + 1) % nl]]),
            ((0, 6), (0, 128 - sdim))).astype(f32)

        body = functools.partial(_edge_kernel, nblk=nblk, sdim=sdim,
                                 vdim=vdim, edim=edim, nb=nb,
                                 layer0=(i == 0))
        e_cur, tbl, svp = pl.pallas_call(
            body,
            grid=(nblk,),
            in_specs=[
                pl.BlockSpec((1, t, 1), lambda i: (i, 0, 0)),
                pl.BlockSpec((1, t, 1), lambda i: (i, 0, 0)),
                pl.BlockSpec((1, 1, t), lambda i: (i, 0, 0)),
                pl.BlockSpec((t, edim), lambda i: (i, 0)),
                pl.BlockSpec((t, 8), lambda i: (i, 0)),
                pl.BlockSpec((n, 128), lambda i: (0, 0)),
                pl.BlockSpec((sdim, 128), lambda i: (0, 0)),
                pl.BlockSpec((sdim, 128), lambda i: (0, 0)),
                pl.BlockSpec((edim, 128), lambda i: (0, 0)),
                pl.BlockSpec((8, 128), lambda i: (0, 0)),
                pl.BlockSpec((sdim, sdim), lambda i: (0, 0)),
                pl.BlockSpec((8, 128), lambda i: (0, 0)),
                pl.BlockSpec((n, 1), lambda i: (0, 0)),
            ],
            out_specs=[
                pl.BlockSpec((t, edim), lambda i: (i, 0)),
                pl.BlockSpec((n, 128), lambda i: (0, 0)),
                pl.BlockSpec((n, 128), lambda i: (0, 0)),
            ],
            out_shape=[
                jax.ShapeDtypeStruct((eg, edim), f32),
                jax.ShapeDtypeStruct((n, 128), f32),
                jax.ShapeDtypeStruct((n, 128), f32),
            ],
            scratch_shapes=[pltpu.VMEM((n, 128), f32)],
        )(src_c, dst_c, dst_r, e_cur, geom0, tbl, w1p, w2p, w3p, w45b,
          W_upd[i].astype(f32), gbn, batch2)

    s_o = svp[:, 0:sdim]
    v_o = svp[:, sdim:sdim + nv].reshape(n, 3, vdim)
    p_o = svp[:, sdim + nv:sdim + nv + 3]
    return s_o, v_o, e_cur, p_o


# speedup vs baseline: 19.3091x; 1.3194x over previous
import functools

import jax
import jax.numpy as jnp
from jax.experimental import pallas as pl
from jax.experimental.pallas import tpu as pltpu

_PREC = jax.lax.Precision.HIGHEST
_HIGH = jax.lax.Precision.HIGHEST
_FAST = jax.lax.Precision.DEFAULT


def _dot(a, b, prec=_PREC):
    return jax.lax.dot_general(a, b, (((1,), (0,)), ((), ())),
                               preferred_element_type=jnp.float32,
                               precision=prec)


def _pre_kernel(s_ref, v_ref, p_ref, g_ref, b_ref, tbl_ref):
    n = s_ref.shape[0]
    s = s_ref[...]
    mu = jnp.mean(s, axis=1, keepdims=True)
    var = jnp.mean((s - mu) * (s - mu), axis=1, keepdims=True)
    sn = (s - mu) / jnp.sqrt(var + 1e-5) * g_ref[...] + b_ref[...]
    v2 = v_ref[...]
    nv = v2.shape[1]
    rms = jnp.sqrt(jnp.sum(v2 * v2, axis=1, keepdims=True) / (nv // 3) + 1e-6)
    vn = v2 / rms
    p3 = p_ref[:, 0:3]
    phi = p3.astype(jnp.bfloat16).astype(jnp.float32)
    plo = p3 - phi
    pad = jnp.zeros((n, 128 - 64 - nv - 6), dtype=jnp.float32)
    tbl_ref[...] = jnp.concatenate([sn, vn, phi, plo, pad], axis=1)


def _edge_kernel(src_c_ref, dst_c_ref, dst_r_ref, e_ref, geom_ref, tbl_ref,
                 w1_ref, w2_ref, w3_ref, w45b_ref, wupd_ref, gb_ref,
                 batch_ref, eout_ref, tbln_ref, svp_ref, acc_ref,
                 *, nblk, sdim, vdim, edim, nb, layer0):
    i = pl.program_id(0)
    n = tbl_ref.shape[0]
    t = e_ref.shape[0]
    nv = 3 * vdim

    tbl = tbl_ref[...]
    src_c = src_c_ref[0]
    dst_c = dst_c_ref[0]
    dst_r = dst_r_ref[0]

    iota_tn = jax.lax.broadcasted_iota(jnp.int16, (t, n), 1)
    ohs_t = (iota_tn == src_c.astype(jnp.int16)).astype(jnp.bfloat16)
    ohd_t = (iota_tn == dst_c.astype(jnp.int16)).astype(jnp.bfloat16)
    iota_nt = jax.lax.broadcasted_iota(jnp.int16, (n, t), 0)
    ohd = (iota_nt == dst_r.astype(jnp.int16)).astype(jnp.bfloat16)

    tblb = tbl.astype(jnp.bfloat16)
    gs = _dot(ohs_t, tblb, _FAST)
    gd = _dot(ohd_t, tblb, _FAST)

    if layer0:
        g8 = geom_ref[...]
        rn = g8[:, 0:3]
        d = g8[:, 3:4]
        a = g8[:, 4:5]
    else:
        ps = gs[:, sdim + nv:sdim + nv + 3] + gs[:, sdim + nv + 3:sdim + nv + 6]
        pd = gd[:, sdim + nv:sdim + nv + 3] + gd[:, sdim + nv + 3:sdim + nv + 6]
        rv = pd - ps
        d = jnp.sqrt(jnp.clip(jnp.sum(rv * rv, axis=1, keepdims=True),
                              1e-6, None))
        a = jnp.sum(ps * pd, axis=1, keepdims=True)
        rn = rv / d

    eb = e_ref[...]
    m = (_dot(gs[:, 0:sdim], w1_ref[...], _FAST)
         + _dot(gd[:, 0:sdim], w2_ref[...], _FAST)
         + _dot(eb, w3_ref[...], _FAST)
         + d * w45b_ref[0:1, :] + a * w45b_ref[1:2, :] + w45b_ref[2:3, :])

    ms = jax.nn.silu(m[:, 0:sdim])
    gv = m[:, sdim:sdim + vdim]
    gr = m[:, sdim + vdim:sdim + 2 * vdim]
    enew = m[:, sdim + 2 * vdim:sdim + 2 * vdim + edim]
    gp = m[:, sdim + 2 * vdim + edim:sdim + 2 * vdim + edim + 1]

    eout_ref[...] = jax.nn.silu(enew)

    vs = gs[:, sdim:sdim + nv]
    gv3 = jnp.concatenate([gv, gv, gv], axis=1)
    gr3 = jnp.concatenate([gr, gr, gr], axis=1)
    rexp = jnp.concatenate(
        [jnp.broadcast_to(rn[:, c:c + 1], (t, vdim)) for c in range(3)],
        axis=1)
    mv = vs * gv3 + rexp * gr3
    pt = rn * jnp.tanh(gp)

    ones = jnp.ones((t, 1), dtype=jnp.float32)
    zpad = jnp.zeros((t, 128 - sdim - nv - 3 - 1), dtype=jnp.float32)
    payload = jnp.concatenate([ms, mv, pt, ones, zpad], axis=1)

    contrib = jax.lax.dot_general(ohd, payload.astype(jnp.bfloat16),
                                  (((1,), (0,)), ((), ())),
                                  preferred_element_type=jnp.float32,
                                  precision=_FAST)

    @pl.when(i == 0)
    def _():
        acc_ref[...] = jnp.zeros_like(acc_ref)

    acc_ref[...] += contrib

    @pl.when(i == nblk - 1)
    def _():
        acc = acc_ref[...]
        ccol = sdim + nv + 3
        cnt = acc[:, ccol:ccol + 1]
        inv = 1.0 / jnp.maximum(cnt, 1.0)
        sn = tbl[:, 0:sdim]
        vn = tbl[:, sdim:sdim + nv]
        pcur = (tbl[:, sdim + nv:sdim + nv + 3]
                + tbl[:, sdim + nv + 3:sdim + nv + 6])
        s_next = sn + _dot(acc[:, 0:sdim] * inv, wupd_ref[...])
        v_next = vn + acc[:, sdim:sdim + nv] * inv
        p_mid = pcur + acc[:, sdim + nv:sdim + nv + 3] * inv

        iota_b = jax.lax.broadcasted_iota(jnp.int32, (n, 128), 1)
        ohb = (iota_b == batch_ref[...]).astype(jnp.float32)
        p4 = jnp.concatenate([p_mid, jnp.ones((n, 1), jnp.float32)], axis=1)
        bs = jax.lax.dot_general(ohb, p4, (((0,), (0,)), ((), ())),
                                 preferred_element_type=jnp.float32,
                                 precision=_PREC)
        minv = 1.0 / jnp.maximum(bs[:, 3:4], 1.0)
        p_next = p_mid - _dot(ohb, bs[:, 0:3] * minv)

        zc = jnp.zeros((n, 128 - sdim - nv - 3), dtype=jnp.float32)
        svp_ref[...] = jnp.concatenate([s_next, v_next, p_next, zc], axis=1)

        mu = jnp.mean(s_next, axis=1, keepdims=True)
        var = jnp.mean((s_next - mu) * (s_next - mu), axis=1, keepdims=True)
        sn2 = ((s_next - mu) / jnp.sqrt(var + 1e-5) * gb_ref[0:1, 0:sdim]
               + gb_ref[1:2, 0:sdim])
        rms = jnp.sqrt(jnp.sum(v_next * v_next, axis=1, keepdims=True)
                       / vdim + 1e-6)
        vn2 = v_next / rms
        phi = p_next.astype(jnp.bfloat16).astype(jnp.float32)
        plo = p_next - phi
        zc2 = jnp.zeros((n, 128 - sdim - nv - 6), dtype=jnp.float32)
        tbln_ref[...] = jnp.concatenate([sn2, vn2, phi, plo, zc2], axis=1)


def _pad_cols(x, w=128):
    return jnp.pad(x, ((0, 0), (0, w - x.shape[1])))


def kernel(s, v, p, edge_index_local, d_local, a_local, r_norm_local,
           e_local, edge_index_global, d_global, a_global, r_norm_global,
           e_global, batch, W_msg, b_msg, W_upd, gamma_s, beta_s):
    n, sdim = s.shape
    vdim = v.shape[2]
    nv = 3 * vdim
    eg = e_global.shape[0]
    edim = e_global.shape[1]
    nl = W_msg.shape[0]
    nb = 16

    t = min(1024, eg)
    nblk = eg // t

    f32 = jnp.float32
    v2 = v.reshape(n, nv).astype(f32)
    p8 = jnp.pad(p.astype(f32), ((0, 0), (0, 5)))
    src = edge_index_global[0].astype(jnp.int32)
    dst = edge_index_global[1].astype(jnp.int32)
    src_c = src.reshape(nblk, t, 1)
    dst_c = dst.reshape(nblk, t, 1)
    dst_r = dst.reshape(nblk, 1, t)
    geom0 = jnp.concatenate(
        [r_norm_global.astype(f32), d_global[:, None].astype(f32),
         a_global[:, None].astype(f32), jnp.zeros((eg, 3), f32)],
        axis=1)

    tbl = pl.pallas_call(
        _pre_kernel,
        out_shape=jax.ShapeDtypeStruct((n, 128), f32),
    )(s.astype(f32), v2, p8, gamma_s[0][None, :].astype(f32),
      beta_s[0][None, :].astype(f32))

    batch2 = batch.reshape(n, 1).astype(jnp.int32)

    e_cur = e_global.astype(f32)
    svp = None
    for i in range(nl):
        W = W_msg[i].astype(f32)
        w1p = _pad_cols(W[0:sdim])
        w2p = _pad_cols(W[sdim:2 * sdim])
        w3p = _pad_cols(W[2 * sdim:2 * sdim + edim])
        w45b = jnp.pad(
            jnp.stack([W[2 * sdim + edim], W[2 * sdim + edim + 1],
                       b_msg[i].astype(f32)]),
            ((0, 5), (0, 128 - W.shape[1])))
        gbn = jnp.pad(
            jnp.stack([gamma_s[(i + 1) % nl], beta_s[(i + 1) % nl]]),
            ((0, 6), (0, 128 - sdim))).astype(f32)

        body = functools.partial(_edge_kernel, nblk=nblk, sdim=sdim,
                                 vdim=vdim, edim=edim, nb=nb,
                                 layer0=(i == 0))
        e_cur, tbl, svp = pl.pallas_call(
            body,
            grid=(nblk,),
            in_specs=[
                pl.BlockSpec((1, t, 1), lambda i: (i, 0, 0)),
                pl.BlockSpec((1, t, 1), lambda i: (i, 0, 0)),
                pl.BlockSpec((1, 1, t), lambda i: (i, 0, 0)),
                pl.BlockSpec((t, edim), lambda i: (i, 0)),
                pl.BlockSpec((t, 8), lambda i: (i, 0)),
                pl.BlockSpec((n, 128), lambda i: (0, 0)),
                pl.BlockSpec((sdim, 128), lambda i: (0, 0)),
                pl.BlockSpec((sdim, 128), lambda i: (0, 0)),
                pl.BlockSpec((edim, 128), lambda i: (0, 0)),
                pl.BlockSpec((8, 128), lambda i: (0, 0)),
                pl.BlockSpec((sdim, sdim), lambda i: (0, 0)),
                pl.BlockSpec((8, 128), lambda i: (0, 0)),
                pl.BlockSpec((n, 1), lambda i: (0, 0)),
            ],
            out_specs=[
                pl.BlockSpec((t, edim), lambda i: (i, 0)),
                pl.BlockSpec((n, 128), lambda i: (0, 0)),
                pl.BlockSpec((n, 128), lambda i: (0, 0)),
            ],
            out_shape=[
                jax.ShapeDtypeStruct((eg, edim), f32),
                jax.ShapeDtypeStruct((n, 128), f32),
                jax.ShapeDtypeStruct((n, 128), f32),
            ],
            scratch_shapes=[pltpu.VMEM((n, 128), f32)],
        )(src_c, dst_c, dst_r, e_cur, geom0, tbl, w1p, w2p, w3p, w45b,
          W_upd[i].astype(f32), gbn, batch2)

    s_o = svp[:, 0:sdim]
    v_o = svp[:, sdim:sdim + nv].reshape(n, 3, vdim)
    p_o = svp[:, sdim + nv:sdim + nv + 3]
    return s_o, v_o, e_cur, p_o
